# baseline (speedup 1.0000x reference)
import functools

import jax
import jax.numpy as jnp
from jax import lax
from jax.experimental import pallas as pl
from jax.experimental.pallas import tpu as pltpu
from jax.experimental.pallas import tpu_sc as plsc

EMBED = 16
L = 16
NC = 2
NS = 16
NW = NC * NS
CH = 16
BLK = 128


def _build_sc_call(B):
  b_per_w = B // NW
  n_rounds = b_per_w // CH
  n_groups = b_per_w // L
  mesh = plsc.VectorSubcoreMesh(
      core_axis_name="c", subcore_axis_name="s",
      num_cores=NC, num_subcores=NS)

  @functools.partial(
      pl.kernel,
      out_type=jax.ShapeDtypeStruct((B,), jnp.float32),
      mesh=mesh,
      compiler_params=pltpu.CompilerParams(
          needs_layout_passes=False, use_tc_tiling_on_sc=True),
      scratch_types=[
          pltpu.VMEM((b_per_w,), jnp.int32),
          pltpu.VMEM((b_per_w,), jnp.int32),
          pltpu.VMEM_SHARED((NS, CH * EMBED, BLK), jnp.float32),
          pltpu.VMEM((CH * EMBED, BLK), jnp.float32),
          pltpu.VMEM((b_per_w * EMBED,), jnp.float32),
          pltpu.VMEM((b_per_w * EMBED,), jnp.float32),
          pltpu.VMEM((b_per_w,), jnp.float32),
          pltpu.VMEM((EMBED, L), jnp.float32),
          pltpu.VMEM((L,), jnp.float32),
          pltpu.SemaphoreType.DMA,
      ],
  )
  def gmf(user_h, item_h, u_t, i_t, w_h, b_h, out_h,
          uidx, iidx, blk_sh, blk, urows, irows, outv, wv, bv, sem):
    sid = lax.axis_index("s")
    wid = sid * NC + lax.axis_index("c")
    base = pl.multiple_of(wid * b_per_w, b_per_w)

    pltpu.sync_copy(user_h.at[pl.ds(base, b_per_w)], uidx)
    pltpu.sync_copy(item_h.at[pl.ds(base, b_per_w)], iidx)
    pltpu.sync_copy(w_h, wv)
    pltpu.sync_copy(b_h, bv)

    iot = lax.iota(jnp.int32, L)
    lanes_mask = jnp.full((L,), BLK - 1, jnp.int32)

    def fetch_round(tab, idx_ref, rows_ref, c):
      c0 = pl.multiple_of(c * CH, CH)
      vecs = [idx_ref[pl.ds(c0 + v * L, L)] for v in range(CH // L)]
      copies = []
      for j in range(CH):
        r = vecs[j // L][j % L]
        col0 = pl.multiple_of(
            lax.shift_left(lax.shift_right_logical(r, 7), 7), BLK)
        copies.append(pltpu.async_copy(
            tab.at[:, pl.ds(col0, BLK)],
            blk_sh.at[sid, pl.ds(j * EMBED, EMBED), :], sem))
      for cp in copies:
        cp.wait()
      pltpu.async_copy(blk_sh.at[sid], blk, sem).wait()
      for v in range(CH // L):
        cols = lax.bitwise_and(vecs[v], lanes_mask)
        for j in range(L):
          el = v * L + j
          rows = iot + el * EMBED
          col = jnp.broadcast_to(cols[j], (L,))
          vec = plsc.load_gather(blk, [rows, col])
          rows_ref[pl.ds((c0 + el) * EMBED, EMBED)] = vec

    def u_round(c, carry):
      fetch_round(u_t, uidx, urows, c)
      return carry

    def i_round(c, carry):
      fetch_round(i_t, iidx, irows, c)
      return carry

    lax.fori_loop(0, n_rounds, u_round, 0)
    lax.fori_loop(0, n_rounds, i_round, 0)

    bias_vec = bv[...]
    wsp = [wv[e] for e in range(EMBED)]
    iot16 = iot * EMBED
    zero = jnp.zeros((L,), jnp.float32)

    def group(g, carry):
      g16 = pl.multiple_of(g * L, L)
      base_i = iot16 + g16 * EMBED
      accs = [bias_vec, zero, zero, zero]
      for e in range(EMBED):
        idx = base_i + e
        uc = plsc.load_gather(urows, [idx])
        ic = plsc.load_gather(irows, [idx])
        accs[e % 4] = accs[e % 4] + (uc * ic) * wsp[e]
      outv[pl.ds(g16, L)] = (accs[0] + accs[1]) + (accs[2] + accs[3])
      return carry

    lax.fori_loop(0, n_groups, group, 0)

    pltpu.sync_copy(outv, out_h.at[pl.ds(base, b_per_w)])

  return gmf


def kernel(user, item, U, I, fc_w, fc_b):
  B = user.shape[0]
  u_t = U.T
  i_t = I.T
  user1 = user.astype(jnp.int32)
  item1 = item.astype(jnp.int32)
  w_vec = jnp.broadcast_to(
      fc_w.reshape(EMBED, 1).astype(jnp.float32), (EMBED, L))
  b_vec = jnp.broadcast_to(fc_b.reshape(()), (L,)).astype(jnp.float32)
  return _build_sc_call(B)(user1, item1, u_t, i_t, w_vec, b_vec)


# speedup vs baseline: 1.4705x; 1.4705x over previous
import functools

import jax
import jax.numpy as jnp
from jax import lax
from jax.experimental import pallas as pl
from jax.experimental.pallas import tpu as pltpu
from jax.experimental.pallas import tpu_sc as plsc

EMBED = 16
L = 16
NC = 2
NS = 16
NW = NC * NS
CH = 32
BLK = 128


def _build_sc_call(B):
  b_per_w = B // NW
  n_rounds = b_per_w // CH
  n_groups = b_per_w // L
  mesh = plsc.VectorSubcoreMesh(
      core_axis_name="c", subcore_axis_name="s",
      num_cores=NC, num_subcores=NS)

  @functools.partial(
      pl.kernel,
      out_type=jax.ShapeDtypeStruct((B,), jnp.float32),
      mesh=mesh,
      compiler_params=pltpu.CompilerParams(
          needs_layout_passes=False, use_tc_tiling_on_sc=True),
      scratch_types=[
          pltpu.VMEM((b_per_w,), jnp.int32),
          pltpu.VMEM((b_per_w,), jnp.int32),
          pltpu.VMEM((CH * EMBED, BLK), jnp.float32),
          pltpu.VMEM((b_per_w * EMBED,), jnp.float32),
          pltpu.VMEM((b_per_w * EMBED,), jnp.float32),
          pltpu.VMEM((b_per_w,), jnp.float32),
          pltpu.VMEM((EMBED, L), jnp.float32),
          pltpu.VMEM((L,), jnp.float32),
          pltpu.SemaphoreType.DMA,
      ],
  )
  def gmf(user_h, item_h, u_t, i_t, w_h, b_h, out_h,
          uidx, iidx, blk, urows, irows, outv, wv, bv, sem):
    wid = lax.axis_index("s") * NC + lax.axis_index("c")
    base = pl.multiple_of(wid * b_per_w, b_per_w)

    pltpu.sync_copy(user_h.at[pl.ds(base, b_per_w)], uidx)
    pltpu.sync_copy(item_h.at[pl.ds(base, b_per_w)], iidx)
    pltpu.sync_copy(w_h, wv)
    pltpu.sync_copy(b_h, bv)

    iot = lax.iota(jnp.int32, L)
    lanes_mask = jnp.full((L,), BLK - 1, jnp.int32)

    def fetch_round(tab, idx_ref, rows_ref, c):
      c0 = pl.multiple_of(c * CH, CH)
      vecs = [idx_ref[pl.ds(c0 + v * L, L)] for v in range(CH // L)]
      copies = []
      for j in range(CH):
        r = vecs[j // L][j % L]
        col0 = pl.multiple_of(
            lax.shift_left(lax.shift_right_logical(r, 7), 7), BLK)
        copies.append(pltpu.async_copy(
            tab.at[:, pl.ds(col0, BLK)],
            blk.at[pl.ds(j * EMBED, EMBED), :], sem))
      for cp in copies:
        cp.wait()
      for v in range(CH // L):
        cols = lax.bitwise_and(vecs[v], lanes_mask)
        for j in range(L):
          el = v * L + j
          rows = iot + el * EMBED
          col = jnp.broadcast_to(cols[j], (L,))
          vec = plsc.load_gather(blk, [rows, col])
          rows_ref[pl.ds((c0 + el) * EMBED, EMBED)] = vec

    def u_round(c, carry):
      fetch_round(u_t, uidx, urows, c)
      return carry

    def i_round(c, carry):
      fetch_round(i_t, iidx, irows, c)
      return carry

    lax.fori_loop(0, n_rounds, u_round, 0)
    lax.fori_loop(0, n_rounds, i_round, 0)

    bias_vec = bv[...]
    wsp = [wv[e] for e in range(EMBED)]
    iot16 = iot * EMBED
    zero = jnp.zeros((L,), jnp.float32)

    def group(g, carry):
      g16 = pl.multiple_of(g * L, L)
      base_i = iot16 + g16 * EMBED
      accs = [bias_vec, zero, zero, zero]
      for e in range(EMBED):
        idx = base_i + e
        uc = plsc.load_gather(urows, [idx])
        ic = plsc.load_gather(irows, [idx])
        accs[e % 4] = accs[e % 4] + (uc * ic) * wsp[e]
      outv[pl.ds(g16, L)] = (accs[0] + accs[1]) + (accs[2] + accs[3])
      return carry

    lax.fori_loop(0, n_groups, group, 0)

    pltpu.sync_copy(outv, out_h.at[pl.ds(base, b_per_w)])

  return gmf


def kernel(user, item, U, I, fc_w, fc_b):
  B = user.shape[0]
  u_t = U.T
  i_t = I.T
  user1 = user.astype(jnp.int32)
  item1 = item.astype(jnp.int32)
  w_vec = jnp.broadcast_to(
      fc_w.reshape(EMBED, 1).astype(jnp.float32), (EMBED, L))
  b_vec = jnp.broadcast_to(fc_b.reshape(()), (L,)).astype(jnp.float32)
  return _build_sc_call(B)(user1, item1, u_t, i_t, w_vec, b_vec)


# speedup vs baseline: 1.5108x; 1.0274x over previous
import functools

import jax
import jax.numpy as jnp
from jax import lax
from jax.experimental import pallas as pl
from jax.experimental.pallas import tpu as pltpu
from jax.experimental.pallas import tpu_sc as plsc

EMBED = 16
L = 16
NC = 2
NS = 16
NW = NC * NS
CH = 16
BLK = 128


def _build_sc_call(B):
  b_per_w = B // NW
  n_rounds = b_per_w // CH
  n_groups = b_per_w // L
  mesh = plsc.VectorSubcoreMesh(
      core_axis_name="c", subcore_axis_name="s",
      num_cores=NC, num_subcores=NS)

  @functools.partial(
      pl.kernel,
      out_type=jax.ShapeDtypeStruct((B,), jnp.float32),
      mesh=mesh,
      compiler_params=pltpu.CompilerParams(
          needs_layout_passes=False, use_tc_tiling_on_sc=True),
      scratch_types=[
          pltpu.VMEM((b_per_w,), jnp.int32),
          pltpu.VMEM((b_per_w,), jnp.int32),
          pltpu.VMEM((2, CH * EMBED, BLK), jnp.float32),
          pltpu.VMEM((b_per_w * EMBED,), jnp.float32),
          pltpu.VMEM((b_per_w * EMBED,), jnp.float32),
          pltpu.VMEM((b_per_w,), jnp.float32),
          pltpu.VMEM((EMBED, L), jnp.float32),
          pltpu.VMEM((L,), jnp.float32),
          pltpu.SemaphoreType.DMA,
          pltpu.SemaphoreType.DMA,
      ],
  )
  def gmf(user_h, item_h, u_t, i_t, w_h, b_h, out_h,
          uidx, iidx, blk, urows, irows, outv, wv, bv, sem0, sem1):
    wid = lax.axis_index("s") * NC + lax.axis_index("c")
    base = pl.multiple_of(wid * b_per_w, b_per_w)

    pltpu.sync_copy(user_h.at[pl.ds(base, b_per_w)], uidx)
    pltpu.sync_copy(item_h.at[pl.ds(base, b_per_w)], iidx)
    pltpu.sync_copy(w_h, wv)
    pltpu.sync_copy(b_h, bv)

    iot = lax.iota(jnp.int32, L)
    lanes_mask = jnp.full((L,), BLK - 1, jnp.int32)

    sems = (sem0, sem1)

    def fire(tab, idx_ref, c, b):
      c0 = pl.multiple_of(c * CH, CH)
      vec = idx_ref[pl.ds(c0, L)]
      for j in range(CH):
        r = vec[j]
        col0 = pl.multiple_of(
            lax.shift_left(lax.shift_right_logical(r, 7), 7), BLK)
        pltpu.async_copy(
            tab.at[:, pl.ds(col0, BLK)],
            blk.at[b, pl.ds(j * EMBED, EMBED), :], sems[b])

    def drain(tab, b):
      for j in range(CH):
        pltpu.make_async_copy(
            tab.at[:, pl.ds(0, BLK)],
            blk.at[b, pl.ds(j * EMBED, EMBED), :], sems[b]).wait()

    def extract(idx_ref, rows_ref, c, b):
      c0 = pl.multiple_of(c * CH, CH)
      cols = lax.bitwise_and(idx_ref[pl.ds(c0, L)], lanes_mask)
      bvec = jnp.full((L,), b, jnp.int32)
      for j in range(CH):
        rows = iot + j * EMBED
        col = jnp.broadcast_to(cols[j], (L,))
        vec = plsc.load_gather(blk, [bvec, rows, col])
        rows_ref[pl.ds((c0 + j) * EMBED, EMBED)] = vec

    def table_pass(tab, idx_ref, rows_ref):
      fire(tab, idx_ref, 0, 0)

      def body(k, carry):
        k2 = k * 2
        fire(tab, idx_ref, k2 + 1, 1)
        drain(tab, 0)
        extract(idx_ref, rows_ref, k2, 0)

        @pl.when(k < n_rounds // 2 - 1)
        def _():
          fire(tab, idx_ref, k2 + 2, 0)

        drain(tab, 1)
        extract(idx_ref, rows_ref, k2 + 1, 1)
        return carry

      lax.fori_loop(0, n_rounds // 2, body, 0)

    table_pass(u_t, uidx, urows)
    table_pass(i_t, iidx, irows)

    bias_vec = bv[...]
    wsp = [wv[e] for e in range(EMBED)]
    iot16 = iot * EMBED
    zero = jnp.zeros((L,), jnp.float32)

    def group(g, carry):
      g16 = pl.multiple_of(g * L, L)
      base_i = iot16 + g16 * EMBED
      accs = [bias_vec, zero, zero, zero]
      for e in range(EMBED):
        idx = base_i + e
        uc = plsc.load_gather(urows, [idx])
        ic = plsc.load_gather(irows, [idx])
        accs[e % 4] = accs[e % 4] + (uc * ic) * wsp[e]
      outv[pl.ds(g16, L)] = (accs[0] + accs[1]) + (accs[2] + accs[3])
      return carry

    lax.fori_loop(0, n_groups, group, 0)

    pltpu.sync_copy(outv, out_h.at[pl.ds(base, b_per_w)])

  return gmf


def kernel(user, item, U, I, fc_w, fc_b):
  B = user.shape[0]
  u_t = U.T
  i_t = I.T
  user1 = user.astype(jnp.int32)
  item1 = item.astype(jnp.int32)
  w_vec = jnp.broadcast_to(
      fc_w.reshape(EMBED, 1).astype(jnp.float32), (EMBED, L))
  b_vec = jnp.broadcast_to(fc_b.reshape(()), (L,)).astype(jnp.float32)
  return _build_sc_call(B)(user1, item1, u_t, i_t, w_vec, b_vec)
